# baseline (speedup 1.0000x reference)
import functools

import jax
import jax.numpy as jnp
from jax import lax
from jax.experimental import pallas as pl
from jax.experimental.pallas import tpu as pltpu

N = 10000
E = 320000
D = 128
H = 8
C = 16
NEG_SLOPE = 0.2
BN_EPS = 1e-5

ROWS = 400
GRID = N // ROWS


def _lrelu(x):
    return jnp.where(x > 0, x, NEG_SLOPE * x)


def _head_body(x_ref, w_ref, m1_ref, m2_ref, h_ref, u_ref, v_ref, avec_ref):
    i = pl.program_id(0)
    h = jnp.dot(x_ref[...], w_ref[...], preferred_element_type=jnp.float32)
    h_ref[...] = h
    u = jnp.dot(h, m1_ref[...], preferred_element_type=jnp.float32)
    v = jnp.dot(h, m2_ref[...], preferred_element_type=jnp.float32)
    u_ref[...] = u
    v_ref[...] = v

    @pl.when(i == 0)
    def _():
        avec_ref[...] = jnp.full((8, 16), -1e30, jnp.float32)

    bm = jnp.max(u[:, :H], axis=0)
    bm16 = jnp.concatenate([bm, jnp.full((8,), 1e30, jnp.float32)])
    avec_ref[...] = jnp.maximum(avec_ref[...], bm16[None, :])


def _head(x, W, M1, M2):
    return pl.pallas_call(
        _head_body,
        grid=(GRID,),
        in_specs=[
            pl.BlockSpec((ROWS, D), lambda i: (i, 0)),
            pl.BlockSpec((D, D), lambda i: (0, 0)),
            pl.BlockSpec((D, 2 * H), lambda i: (0, 0)),
            pl.BlockSpec((D, 2 * H), lambda i: (0, 0)),
        ],
        out_specs=[
            pl.BlockSpec((ROWS, D), lambda i: (i, 0)),
            pl.BlockSpec((ROWS, 2 * H), lambda i: (i, 0)),
            pl.BlockSpec((ROWS, 2 * H), lambda i: (i, 0)),
            pl.BlockSpec((8, 16), lambda i: (0, 0)),
        ],
        out_shape=[
            jax.ShapeDtypeStruct((N, D), jnp.float32),
            jax.ShapeDtypeStruct((N, 2 * H), jnp.float32),
            jax.ShapeDtypeStruct((N, 2 * H), jnp.float32),
            jax.ShapeDtypeStruct((8, 16), jnp.float32),
        ],
    )(x, W, M1, M2)


def _combine_body(p0_ref, p1_ref, b_ref, g_ref, st_ref):
    i = pl.program_id(0)
    g = p0_ref[...] + p1_ref[...] + b_ref[...]
    g_ref[...] = g
    s1 = jnp.sum(g, axis=0)
    s2 = jnp.sum(g * g, axis=0)
    blk = jnp.concatenate(
        [s1[None, :], s2[None, :], jnp.zeros((6, D), jnp.float32)], axis=0)

    @pl.when(i == 0)
    def _():
        st_ref[...] = jnp.zeros((8, D), jnp.float32)

    st_ref[...] = st_ref[...] + blk


def _combine_stats(p0, p1, b):
    return pl.pallas_call(
        _combine_body,
        grid=(GRID,),
        in_specs=[
            pl.BlockSpec((ROWS, D), lambda i: (i, 0)),
            pl.BlockSpec((ROWS, D), lambda i: (i, 0)),
            pl.BlockSpec((1, D), lambda i: (0, 0)),
        ],
        out_specs=[
            pl.BlockSpec((ROWS, D), lambda i: (i, 0)),
            pl.BlockSpec((8, D), lambda i: (0, 0)),
        ],
        out_shape=[
            jax.ShapeDtypeStruct((N, D), jnp.float32),
            jax.ShapeDtypeStruct((8, D), jnp.float32),
        ],
    )(p0, p1, b)


def _bn_act(g, st, gamma, beta):
    mu = st[0:1, :] * (1.0 / N)
    var = st[1:2, :] * (1.0 / N) - mu * mu
    inv = lax.rsqrt(var + BN_EPS)
    a = (g - mu) * inv * gamma + beta
    return jnp.where(a > 0, a, jnp.exp(a) - 1.0)


def _bn_head_body(g_ref, st_ref, gamma_ref, beta_ref, w_ref, m1_ref, m2_ref,
                  h_ref, u_ref, v_ref, avec_ref):
    i = pl.program_id(0)
    act = _bn_act(g_ref[...], st_ref[...], gamma_ref[...], beta_ref[...])
    h = jnp.dot(act, w_ref[...], preferred_element_type=jnp.float32)
    h_ref[...] = h
    u = jnp.dot(h, m1_ref[...], preferred_element_type=jnp.float32)
    v = jnp.dot(h, m2_ref[...], preferred_element_type=jnp.float32)
    u_ref[...] = u
    v_ref[...] = v

    @pl.when(i == 0)
    def _():
        avec_ref[...] = jnp.full((8, 16), -1e30, jnp.float32)

    bm = jnp.max(u[:, :H], axis=0)
    bm16 = jnp.concatenate([bm, jnp.full((8,), 1e30, jnp.float32)])
    avec_ref[...] = jnp.maximum(avec_ref[...], bm16[None, :])


def _bn_head(g, st, gamma, beta, W, M1, M2):
    return pl.pallas_call(
        _bn_head_body,
        grid=(GRID,),
        in_specs=[
            pl.BlockSpec((ROWS, D), lambda i: (i, 0)),
            pl.BlockSpec((8, D), lambda i: (0, 0)),
            pl.BlockSpec((1, D), lambda i: (0, 0)),
            pl.BlockSpec((1, D), lambda i: (0, 0)),
            pl.BlockSpec((D, D), lambda i: (0, 0)),
            pl.BlockSpec((D, 2 * H), lambda i: (0, 0)),
            pl.BlockSpec((D, 2 * H), lambda i: (0, 0)),
        ],
        out_specs=[
            pl.BlockSpec((ROWS, D), lambda i: (i, 0)),
            pl.BlockSpec((ROWS, 2 * H), lambda i: (i, 0)),
            pl.BlockSpec((ROWS, 2 * H), lambda i: (i, 0)),
            pl.BlockSpec((8, 16), lambda i: (0, 0)),
        ],
        out_shape=[
            jax.ShapeDtypeStruct((N, D), jnp.float32),
            jax.ShapeDtypeStruct((N, 2 * H), jnp.float32),
            jax.ShapeDtypeStruct((N, 2 * H), jnp.float32),
            jax.ShapeDtypeStruct((8, 16), jnp.float32),
        ],
    )(g, st, gamma, beta, W, M1, M2)


def _bn_out_body(g_ref, st_ref, gamma_ref, beta_ref, o_ref):
    o_ref[...] = _bn_act(g_ref[...], st_ref[...], gamma_ref[...], beta_ref[...])


def _bn_out(g, st, gamma, beta):
    return pl.pallas_call(
        _bn_out_body,
        grid=(GRID,),
        in_specs=[
            pl.BlockSpec((ROWS, D), lambda i: (i, 0)),
            pl.BlockSpec((8, D), lambda i: (0, 0)),
            pl.BlockSpec((1, D), lambda i: (0, 0)),
            pl.BlockSpec((1, D), lambda i: (0, 0)),
        ],
        out_specs=pl.BlockSpec((ROWS, D), lambda i: (i, 0)),
        out_shape=jax.ShapeDtypeStruct((N, D), jnp.float32),
    )(g, st, gamma, beta)


def _edge_phase(h, U, V, Avec, edge_index):
    src, dst = edge_index[0], edge_index[1]
    A = Avec[0, :H]
    e = _lrelu(U[src, :H] + V[dst, :H])
    m = _lrelu(A[None, :] + V[dst, :H])
    ex = jnp.exp(e - m)
    s = jax.ops.segment_sum(ex, dst, num_segments=N)
    alpha = ex / (s[dst] + 1e-16)
    msg = (h[src].reshape(E, H, C) * alpha[:, :, None]).reshape(E, D)
    out = jax.ops.segment_sum(msg, dst, num_segments=N)
    half = out * 0.5
    return half, half


def _layer_tables(att_src, att_dst):
    M1 = jnp.concatenate(
        [jax.scipy.linalg.block_diag(*[att_src[i][:, None] for i in range(H)]),
         jax.scipy.linalg.block_diag(*[att_dst[i][:, None] for i in range(H)])],
        axis=1)
    M2 = jnp.concatenate([M1[:, H:], M1[:, :H]], axis=1)
    return M1, M2


def kernel(x, edge_index, W1, att_src1, att_dst1, b1, gamma1, beta1,
           W2, att_src2, att_dst2, b2, gamma2, beta2):
    M11, M21 = _layer_tables(att_src1, att_dst1)
    M12, M22 = _layer_tables(att_src2, att_dst2)

    h1, U1, V1, A1 = _head(x, W1, M11, M21)
    p0, p1 = _edge_phase(h1, U1, V1, A1, edge_index)
    g1, st1 = _combine_stats(p0, p1, b1[None, :])

    h2, U2, V2, A2 = _bn_head(g1, st1, gamma1[None, :], beta1[None, :],
                              W2, M12, M22)
    q0, q1 = _edge_phase(h2, U2, V2, A2, edge_index)
    g2, st2 = _combine_stats(q0, q1, b2[None, :])
    return _bn_out(g2, st2, gamma2[None, :], beta2[None, :])


# speedup vs baseline: 91.8232x; 91.8232x over previous
import functools

import jax
import jax.numpy as jnp
from jax import lax
from jax.experimental import pallas as pl
from jax.experimental.pallas import tpu as pltpu
from jax.experimental.pallas import tpu_sc as plsc

N = 10000
E = 320000
D = 128
H = 8
C = 16
NEG_SLOPE = 0.2
BN_EPS = 1e-5

ROWS = 400
GRID = N // ROWS

NC = 2
NS = 16
NW = NC * NS
EPT = E // NW
K = 80
NCHUNK = EPT // K
NP = 10240
RPT = NP // NS


def _lrelu(x):
    return jnp.where(x > 0, x, NEG_SLOPE * x)


def _head_body(x_ref, w_ref, m1_ref, m2_ref, h_ref, u_ref, v_ref, avec_ref):
    i = pl.program_id(0)
    h = jnp.dot(x_ref[...], w_ref[...], preferred_element_type=jnp.float32)
    h_ref[...] = h
    u = jnp.dot(h, m1_ref[...], preferred_element_type=jnp.float32)
    v = jnp.dot(h, m2_ref[...], preferred_element_type=jnp.float32)
    u_ref[...] = u
    v_ref[...] = v

    @pl.when(i == 0)
    def _():
        avec_ref[...] = jnp.full((8, 16), -1e30, jnp.float32)

    bm = jnp.max(u[:, :H], axis=0)
    bm16 = jnp.concatenate([bm, jnp.full((8,), 1e30, jnp.float32)])
    avec_ref[...] = jnp.maximum(avec_ref[...], bm16[None, :])


def _head(x, W, M1, M2):
    return pl.pallas_call(
        _head_body,
        grid=(GRID,),
        in_specs=[
            pl.BlockSpec((ROWS, D), lambda i: (i, 0)),
            pl.BlockSpec((D, D), lambda i: (0, 0)),
            pl.BlockSpec((D, 2 * H), lambda i: (0, 0)),
            pl.BlockSpec((D, 2 * H), lambda i: (0, 0)),
        ],
        out_specs=[
            pl.BlockSpec((ROWS, D), lambda i: (i, 0)),
            pl.BlockSpec((ROWS, 2 * H), lambda i: (i, 0)),
            pl.BlockSpec((ROWS, 2 * H), lambda i: (i, 0)),
            pl.BlockSpec((8, 16), lambda i: (0, 0)),
        ],
        out_shape=[
            jax.ShapeDtypeStruct((N, D), jnp.float32),
            jax.ShapeDtypeStruct((N, 2 * H), jnp.float32),
            jax.ShapeDtypeStruct((N, 2 * H), jnp.float32),
            jax.ShapeDtypeStruct((8, 16), jnp.float32),
        ],
    )(x, W, M1, M2)


def _combine_body(p0_ref, p1_ref, b_ref, g_ref, st_ref):
    i = pl.program_id(0)
    g = jnp.concatenate([p0_ref[...], p1_ref[...]], axis=1) + b_ref[...]
    g_ref[...] = g
    s1 = jnp.sum(g, axis=0)
    s2 = jnp.sum(g * g, axis=0)
    blk = jnp.concatenate(
        [s1[None, :], s2[None, :], jnp.zeros((6, D), jnp.float32)], axis=0)

    @pl.when(i == 0)
    def _():
        st_ref[...] = jnp.zeros((8, D), jnp.float32)

    st_ref[...] = st_ref[...] + blk


def _combine_stats(p0, p1, b):
    return pl.pallas_call(
        _combine_body,
        grid=(GRID,),
        in_specs=[
            pl.BlockSpec((ROWS, D // 2), lambda i: (i, 0)),
            pl.BlockSpec((ROWS, D // 2), lambda i: (i, 0)),
            pl.BlockSpec((1, D), lambda i: (0, 0)),
        ],
        out_specs=[
            pl.BlockSpec((ROWS, D), lambda i: (i, 0)),
            pl.BlockSpec((8, D), lambda i: (0, 0)),
        ],
        out_shape=[
            jax.ShapeDtypeStruct((N, D), jnp.float32),
            jax.ShapeDtypeStruct((8, D), jnp.float32),
        ],
    )(p0, p1, b)


def _bn_act(g, st, gamma, beta):
    mu = st[0:1, :] * (1.0 / N)
    var = st[1:2, :] * (1.0 / N) - mu * mu
    inv = lax.rsqrt(var + BN_EPS)
    a = (g - mu) * inv * gamma + beta
    return jnp.where(a > 0, a, jnp.exp(a) - 1.0)


def _bn_head_body(g_ref, st_ref, gamma_ref, beta_ref, w_ref, m1_ref, m2_ref,
                  h_ref, u_ref, v_ref, avec_ref):
    i = pl.program_id(0)
    act = _bn_act(g_ref[...], st_ref[...], gamma_ref[...], beta_ref[...])
    h = jnp.dot(act, w_ref[...], preferred_element_type=jnp.float32)
    h_ref[...] = h
    u = jnp.dot(h, m1_ref[...], preferred_element_type=jnp.float32)
    v = jnp.dot(h, m2_ref[...], preferred_element_type=jnp.float32)
    u_ref[...] = u
    v_ref[...] = v

    @pl.when(i == 0)
    def _():
        avec_ref[...] = jnp.full((8, 16), -1e30, jnp.float32)

    bm = jnp.max(u[:, :H], axis=0)
    bm16 = jnp.concatenate([bm, jnp.full((8,), 1e30, jnp.float32)])
    avec_ref[...] = jnp.maximum(avec_ref[...], bm16[None, :])


def _bn_head(g, st, gamma, beta, W, M1, M2):
    return pl.pallas_call(
        _bn_head_body,
        grid=(GRID,),
        in_specs=[
            pl.BlockSpec((ROWS, D), lambda i: (i, 0)),
            pl.BlockSpec((8, D), lambda i: (0, 0)),
            pl.BlockSpec((1, D), lambda i: (0, 0)),
            pl.BlockSpec((1, D), lambda i: (0, 0)),
            pl.BlockSpec((D, D), lambda i: (0, 0)),
            pl.BlockSpec((D, 2 * H), lambda i: (0, 0)),
            pl.BlockSpec((D, 2 * H), lambda i: (0, 0)),
        ],
        out_specs=[
            pl.BlockSpec((ROWS, D), lambda i: (i, 0)),
            pl.BlockSpec((ROWS, 2 * H), lambda i: (i, 0)),
            pl.BlockSpec((ROWS, 2 * H), lambda i: (i, 0)),
            pl.BlockSpec((8, 16), lambda i: (0, 0)),
        ],
        out_shape=[
            jax.ShapeDtypeStruct((N, D), jnp.float32),
            jax.ShapeDtypeStruct((N, 2 * H), jnp.float32),
            jax.ShapeDtypeStruct((N, 2 * H), jnp.float32),
            jax.ShapeDtypeStruct((8, 16), jnp.float32),
        ],
    )(g, st, gamma, beta, W, M1, M2)


def _bn_out_body(g_ref, st_ref, gamma_ref, beta_ref, o_ref):
    o_ref[...] = _bn_act(g_ref[...], st_ref[...], gamma_ref[...], beta_ref[...])


def _bn_out(g, st, gamma, beta):
    return pl.pallas_call(
        _bn_out_body,
        grid=(GRID,),
        in_specs=[
            pl.BlockSpec((ROWS, D), lambda i: (i, 0)),
            pl.BlockSpec((8, D), lambda i: (0, 0)),
            pl.BlockSpec((1, D), lambda i: (0, 0)),
            pl.BlockSpec((1, D), lambda i: (0, 0)),
        ],
        out_specs=pl.BlockSpec((ROWS, D), lambda i: (i, 0)),
        out_shape=jax.ShapeDtypeStruct((N, D), jnp.float32),
    )(g, st, gamma, beta)


_SC_MESH = plsc.VectorSubcoreMesh(
    core_axis_name="c", subcore_axis_name="s", num_cores=NC, num_subcores=NS)


def _lane_splat(vec, t):
    idx = jnp.full((16, 1), t, jnp.int32)
    dn = lax.GatherDimensionNumbers(
        offset_dims=(), collapsed_slice_dims=(0,), start_index_map=(0,))
    return lax.gather(vec, idx, dn, (1,),
                      mode=lax.GatherScatterMode.PROMISE_IN_BOUNDS)


@functools.partial(
    pl.kernel,
    out_type=[
        jax.ShapeDtypeStruct((E, 16), jnp.float32),
        jax.ShapeDtypeStruct((NC * NP, 16), jnp.float32),
    ],
    mesh=_SC_MESH,
    compiler_params=pltpu.CompilerParams(use_tc_tiling_on_sc=False),
    scratch_types=[
        pltpu.VMEM((K,), jnp.int32),
        pltpu.VMEM((K,), jnp.int32),
        pltpu.VMEM((K, 16), jnp.float32),
        pltpu.VMEM((K, 16), jnp.float32),
        pltpu.VMEM((K, 16), jnp.float32),
        pltpu.VMEM((16,), jnp.float32),
        pltpu.VMEM((RPT, 16), jnp.float32),
        pltpu.VMEM_SHARED((NP, 16), jnp.float32),
        pltpu.SemaphoreType.DMA,
        pltpu.SemaphoreType.DMA,
    ],
)
def _sc_pass1(u_hbm, v_hbm, avec_hbm, src_hbm, dst_hbm, ex_hbm, sp_hbm,
              src_v, dst_v, u_v, v_v, ex_v, a_v, stage_v, s_sh, sem1, sem2):
    cid = lax.axis_index("c")
    sid = lax.axis_index("s")
    wid = sid * NC + cid

    z16 = jnp.zeros((16,), jnp.float32)

    def zero_body(r, _):
        stage_v[r, :] = z16
        return 0

    lax.fori_loop(0, RPT, zero_body, 0)
    pltpu.sync_copy(stage_v, s_sh.at[pl.ds(sid * RPT, RPT)])
    plsc.subcore_barrier()

    pltpu.sync_copy(avec_hbm.at[0], a_v)
    a16 = a_v[...]

    def chunk_body(i, _):
        base = wid * EPT + i * K
        pltpu.sync_copy(src_hbm.at[pl.ds(base, K)], src_v)
        pltpu.sync_copy(dst_hbm.at[pl.ds(base, K)], dst_v)
        pltpu.async_copy(u_hbm.at[src_v], u_v, sem1).wait()
        pltpu.async_copy(v_hbm.at[dst_v], v_v, sem2).wait()

        def edge_body(j, _):
            u16 = u_v[j, :]
            v16 = v_v[j, :]
            e = u16 + v16
            e = jnp.where(e > 0, e, NEG_SLOPE * e)
            m = a16 + v16
            m = jnp.where(m > 0, m, NEG_SLOPE * m)
            ex_v[j, :] = jnp.exp(e - m)
            return 0

        lax.fori_loop(0, K, edge_body, 0)
        pltpu.sync_copy(ex_v, ex_hbm.at[pl.ds(base, K)])
        pltpu.sync_copy(ex_v, s_sh.at[dst_v], add=True)
        return 0

    lax.fori_loop(0, NCHUNK, chunk_body, 0)

    plsc.subcore_barrier()
    pltpu.sync_copy(s_sh.at[pl.ds(sid * RPT, RPT)], stage_v)
    pltpu.sync_copy(stage_v, sp_hbm.at[pl.ds(cid * NP + sid * RPT, RPT)])


_ZROWS = 128
HD = D // NC
EPT2 = E // NS
NCHUNK2 = EPT2 // K


@functools.partial(
    pl.kernel,
    out_type=jax.ShapeDtypeStruct((NC * NP, HD), jnp.float32),
    mesh=_SC_MESH,
    compiler_params=pltpu.CompilerParams(use_tc_tiling_on_sc=False),
    scratch_types=[
        pltpu.VMEM((K,), jnp.int32),
        pltpu.VMEM((K,), jnp.int32),
        pltpu.VMEM((K, 16), jnp.float32),
        pltpu.VMEM((K, 16), jnp.float32),
        pltpu.VMEM((K, HD), jnp.float32),
        pltpu.VMEM((RPT, 16), jnp.float32),
        pltpu.VMEM((RPT, 16), jnp.float32),
        pltpu.VMEM((_ZROWS, HD), jnp.float32),
        pltpu.VMEM_SHARED((NP, 16), jnp.float32),
        pltpu.VMEM_SHARED((NP, HD), jnp.float32),
        pltpu.SemaphoreType.DMA,
        pltpu.SemaphoreType.DMA,
    ],
)
def _sc_pass2(h2_hbm, ex_hbm, sp_hbm, src_hbm, dst_hbm, op_hbm,
              src_v, dst_v, ex_v, srow_v, h_v, sb0, sb1, zbuf,
              s_sh, out_sh, sem1, sem2):
    cid = lax.axis_index("c")
    sid = lax.axis_index("s")

    pltpu.sync_copy(sp_hbm.at[pl.ds(sid * RPT, RPT)], sb0)
    pltpu.sync_copy(sp_hbm.at[pl.ds(NP + sid * RPT, RPT)], sb1)

    def sum_body(r, _):
        sb0[r, :] = sb0[r, :] + sb1[r, :]
        return 0

    lax.fori_loop(0, RPT, sum_body, 0)
    pltpu.sync_copy(sb0, s_sh.at[pl.ds(sid * RPT, RPT)])

    z16 = jnp.zeros((16,), jnp.float32)

    def zero_body(r, _):
        def zlane(t, _):
            zbuf[r, pl.ds(t * 16, 16)] = z16
            return 0
        lax.fori_loop(0, HD // 16, zlane, 0)
        return 0

    lax.fori_loop(0, _ZROWS, zero_body, 0)
    for t in range(RPT // _ZROWS):
        pltpu.sync_copy(
            zbuf, out_sh.at[pl.ds(sid * RPT + t * _ZROWS, _ZROWS)])
    plsc.subcore_barrier()

    def chunk_body(i, _):
        base = sid * EPT2 + i * K
        pltpu.sync_copy(src_hbm.at[pl.ds(base, K)], src_v)
        pltpu.sync_copy(dst_hbm.at[pl.ds(base, K)], dst_v)
        pltpu.sync_copy(ex_hbm.at[pl.ds(base, K)], ex_v)
        pltpu.async_copy(s_sh.at[dst_v], srow_v, sem1).wait()

        def adj_body(j, _):
            sv = src_v[pl.ds(j * 16, 16)]
            src_v[pl.ds(j * 16, 16)] = sv * 2 + cid
            return 0

        lax.fori_loop(0, K // 16, adj_body, 0)
        pltpu.async_copy(h2_hbm.at[src_v], h_v, sem2).wait()

        def edge_body(j, _):
            alpha = ex_v[j, :] / (srow_v[j, :] + 1e-16)
            for t in range(H // NC):
                aspl = _lane_splat(alpha, cid * (H // NC) + t)
                h_v[j, pl.ds(t * 16, 16)] = h_v[j, pl.ds(t * 16, 16)] * aspl
            return 0

        lax.fori_loop(0, K, edge_body, 0)
        pltpu.sync_copy(h_v, out_sh.at[dst_v], add=True)
        return 0

    lax.fori_loop(0, NCHUNK2, chunk_body, 0)

    plsc.subcore_barrier()
    for t in range(RPT // _ZROWS):
        pltpu.sync_copy(
            out_sh.at[pl.ds(sid * RPT + t * _ZROWS, _ZROWS)], zbuf)
        pltpu.sync_copy(
            zbuf, op_hbm.at[pl.ds(cid * NP + sid * RPT + t * _ZROWS, _ZROWS)])


def _edge_phase(h, U, V, Avec, src, dst):
    ex, sp = _sc_pass1(U, V, Avec, src, dst)
    op = _sc_pass2(h.reshape(2 * N, HD), ex, sp, src, dst)
    return op[:N], op[NP:NP + N]


def _layer_tables(att_src, att_dst):
    M1 = jnp.concatenate(
        [jax.scipy.linalg.block_diag(*[att_src[i][:, None] for i in range(H)]),
         jax.scipy.linalg.block_diag(*[att_dst[i][:, None] for i in range(H)])],
        axis=1)
    M2 = jnp.concatenate([M1[:, H:], M1[:, :H]], axis=1)
    return M1, M2


def kernel(x, edge_index, W1, att_src1, att_dst1, b1, gamma1, beta1,
           W2, att_src2, att_dst2, b2, gamma2, beta2):
    M11, M21 = _layer_tables(att_src1, att_dst1)
    M12, M22 = _layer_tables(att_src2, att_dst2)
    src = edge_index[0]
    dst = edge_index[1]

    h1, U1, V1, A1 = _head(x, W1, M11, M21)
    p0, p1 = _edge_phase(h1, U1, V1, A1, src, dst)
    g1, st1 = _combine_stats(p0, p1, b1[None, :])

    h2, U2, V2, A2 = _bn_head(g1, st1, gamma1[None, :], beta1[None, :],
                              W2, M12, M22)
    q0, q1 = _edge_phase(h2, U2, V2, A2, src, dst)
    g2, st2 = _combine_stats(q0, q1, b2[None, :])
    return _bn_out(g2, st2, gamma2[None, :], beta2[None, :])
